# baseline (speedup 1.0000x reference)
import functools

import jax
import jax.numpy as jnp
from jax import lax
from jax.experimental import pallas as pl
from jax.experimental.pallas import tpu as pltpu
from jax.experimental.pallas import tpu_sc as plsc

N = 50000
ENC = 1024
HID = 512
PROJ = 256
NCLS = 2
NINST = 50

TILE = 4096
NSTEPS = (N + TILE - 1) // TILE
APAD = NSTEPS * TILE

GATHER_B = 256


def _stage1_body(x_ref, wfct_ref, bfc_ref, wat_ref, ba_ref, wbt_ref, bb_ref,
                 wc_ref, bc_ref, wbag_ref, bbag_ref,
                 a_out_ref, logits_ref, prob_ref, yhat_ref,
                 m_ref, z_ref, macc_ref):
    i = pl.program_id(0)

    @pl.when(i == 0)
    def _():
        m_ref[...] = jnp.full((NCLS, 1), -jnp.inf, jnp.float32)
        z_ref[...] = jnp.zeros((NCLS, 1), jnp.float32)
        macc_ref[...] = jnp.zeros((NCLS, HID), jnp.float32)

    ct = (((1,), (1,)), ((), ()))
    h = jnp.maximum(
        lax.dot_general(x_ref[...], wfct_ref[...], ct,
                        preferred_element_type=jnp.float32)
        + bfc_ref[...], 0.0)
    a = jnp.tanh(
        lax.dot_general(h, wat_ref[...], ct,
                        preferred_element_type=jnp.float32)
        + ba_ref[...])
    g = jax.nn.sigmoid(
        lax.dot_general(h, wbt_ref[...], ct,
                        preferred_element_type=jnp.float32)
        + bb_ref[...])
    ag = a * g
    a_t = lax.dot_general(wc_ref[...], ag, (((1,), (1,)), ((), ())),
                          preferred_element_type=jnp.float32) + bc_ref[...]
    a_out_ref[...] = a_t

    base = i * TILE
    cmask = (lax.broadcasted_iota(jnp.int32, (NCLS, TILE), 1) + base) < N
    rmask = (lax.broadcasted_iota(jnp.int32, (TILE, 1), 0) + base) < N
    am = jnp.where(cmask, a_t, -jnp.inf)
    hm = jnp.where(rmask, h, 0.0)

    t_max = jnp.max(am, axis=1, keepdims=True)
    m_old = m_ref[...]
    m_new = jnp.maximum(m_old, t_max)
    alpha = jnp.exp(m_old - m_new)
    p = jnp.exp(am - m_new)
    z_ref[...] = z_ref[...] * alpha + jnp.sum(p, axis=1, keepdims=True)
    contrib = lax.dot_general(p, hm, (((1,), (0,)), ((), ())),
                              preferred_element_type=jnp.float32)
    macc_ref[...] = macc_ref[...] * alpha + contrib
    m_ref[...] = m_new

    @pl.when(i == NSTEPS - 1)
    def _():
        mn = macc_ref[...] / z_ref[...]
        bag = jnp.sum(mn * wbag_ref[...], axis=1, keepdims=True) + bbag_ref[...]
        logits_ref[...] = bag
        mx = jnp.max(bag, axis=0, keepdims=True)
        e = jnp.exp(bag - mx)
        prob_ref[...] = e / jnp.sum(e, axis=0, keepdims=True)
        yhat_ref[...] = jnp.where(bag[1:2, 0:1] > bag[0:1, 0:1], 1, 0
                                  ).astype(jnp.int32)


def _stage1(x, w_fct, b_fc2, wat, ba2, wbt, bb2, wc, bc2, wbag, bbag2):
    full = lambda shape: pl.BlockSpec(shape, lambda i: (0, 0))
    return pl.pallas_call(
        _stage1_body,
        grid=(NSTEPS,),
        in_specs=[
            pl.BlockSpec((TILE, ENC), lambda i: (i, 0)),
            full((HID, ENC)), full((1, HID)),
            full((PROJ, HID)), full((1, PROJ)),
            full((PROJ, HID)), full((1, PROJ)),
            full((NCLS, PROJ)), full((NCLS, 1)),
            full((NCLS, HID)), full((NCLS, 1)),
        ],
        out_specs=[
            pl.BlockSpec((NCLS, TILE), lambda i: (0, i)),
            full((NCLS, 1)), full((NCLS, 1)), full((1, 1)),
        ],
        out_shape=[
            jax.ShapeDtypeStruct((NCLS, APAD), jnp.float32),
            jax.ShapeDtypeStruct((NCLS, 1), jnp.float32),
            jax.ShapeDtypeStruct((NCLS, 1), jnp.float32),
            jax.ShapeDtypeStruct((1, 1), jnp.int32),
        ],
        scratch_shapes=[
            pltpu.VMEM((NCLS, 1), jnp.float32),
            pltpu.VMEM((NCLS, 1), jnp.float32),
            pltpu.VMEM((NCLS, HID), jnp.float32),
        ],
    )(x, w_fct, b_fc2, wat, ba2, wbt, bb2, wc, bc2, wbag, bbag2)


def _radix_body(a_ref, thr_ref):
    HI = jnp.uint32(0x80000000)
    av = a_ref[...]
    b = lax.bitcast_convert_type(av, jnp.uint32)
    keyt = jnp.where(b >= HI, ~b, b ^ HI)
    keys = jnp.concatenate([keyt, ~keyt], axis=0)
    half = N // 2
    keys8 = jnp.concatenate([keys[:, :half], keys[:, half:]], axis=0)

    def body(t, p):
        bit = (31 - t).astype(jnp.uint32)
        cand = p | lax.shift_left(jnp.uint32(1), bit)
        cand8 = jnp.concatenate([cand, cand], axis=0)
        cnt8 = jnp.sum((keys8 >= cand8).astype(jnp.int32), axis=1,
                       keepdims=True)
        cnt = cnt8[:2 * NCLS] + cnt8[2 * NCLS:]
        return jnp.where(cnt >= NINST, cand, p)

    p = lax.fori_loop(0, 32, body, jnp.zeros((2 * NCLS, 1), jnp.uint32))
    rowi = lax.broadcasted_iota(jnp.int32, (2 * NCLS, 1), 0)
    ktop = jnp.where(rowi >= NCLS, ~p, p)
    bits = jnp.where(ktop >= HI, ktop ^ HI, ~ktop)
    tf = lax.bitcast_convert_type(bits, jnp.float32)
    thr_ref[...] = jnp.broadcast_to(tf, (2 * NCLS, 16))


def _radix(a2n):
    return pl.pallas_call(
        _radix_body,
        out_shape=jax.ShapeDtypeStruct((2 * NCLS, 16), jnp.float32),
    )(a2n)


SC_CH = 6272
SC_CH_LAST = N - 7 * SC_CH
SC_NV = SC_CH // 16


def _make_sc_select_gather():
    mesh = plsc.VectorSubcoreMesh(core_axis_name="c", subcore_axis_name="s")

    @functools.partial(
        pl.kernel, mesh=mesh,
        out_type=jax.ShapeDtypeStruct((GATHER_B, ENC), jnp.float32),
        compiler_params=pltpu.CompilerParams(needs_layout_passes=False),
        scratch_types=[
            pltpu.VMEM((SC_CH,), jnp.float32),
            pltpu.VMEM((16,), jnp.float32),
            pltpu.VMEM((64,), jnp.int32),
            pltpu.VMEM((16,), jnp.int32),
            pltpu.VMEM((512,), jnp.int32),
            pltpu.VMEM((128,), jnp.int32),
            pltpu.VMEM((64,), jnp.int32),
            pltpu.VMEM((64, ENC), jnp.float32),
            pltpu.VMEM_SHARED((1024,), jnp.int32),
            pltpu.VMEM_SHARED((256,), jnp.int32),
            pltpu.SemaphoreType.DMA,
        ],
    )
    def sc_sel(a_hbm, thr_hbm, x_hbm, out_hbm, a_v, thr_v, buf_v, cnt_v,
               mb_v, mc_v, gbuf_v, rows_v, sh_idx, sh_cnt, sem):
        c = lax.axis_index("c")
        s = lax.axis_index("s")
        dir_ = s // 8
        sub = s % 8
        base = sub * SC_CH
        r_thr = dir_ * NCLS + c
        a_off = pl.multiple_of(c * N + base, 8)

        @pl.when(sub < 7)
        def _():
            pltpu.sync_copy(a_hbm.at[pl.ds(a_off, SC_CH)], a_v)

        @pl.when(sub == 7)
        def _():
            pltpu.sync_copy(a_hbm.at[pl.ds(a_off, SC_CH_LAST)],
                            a_v.at[pl.ds(0, SC_CH_LAST)])

        pltpu.sync_copy(thr_hbm.at[pl.ds(pl.multiple_of(r_thr * 16, 8), 16)],
                        thr_v)
        is_bot = dir_ == 1

        def body(j, cnt):
            v = a_v[pl.ds(j * 16, 16)]
            tf = thr_v[...]
            gidx = base + j * 16 + lax.iota(jnp.int32, 16)
            selm = jnp.where(is_bot, v <= tf, v >= tf)
            m = selm & (gidx < N)
            vi = jnp.where(m, jnp.int32(1), jnp.int32(0))
            csum = plsc.cumsum(vi)
            pos = cnt + csum - 1
            m2 = m & (pos < 64)
            plsc.store_scatter(buf_v, [pos], gidx, mask=m2)
            return cnt + csum[15]

        cntf = lax.fori_loop(0, SC_NV, body, jnp.int32(0), unroll=8)

        cnt_v[...] = jnp.full((16,), cntf, jnp.int32)
        pltpu.sync_copy(cnt_v, sh_cnt.at[pl.ds(pl.multiple_of(s * 16, 8), 16)])
        pltpu.sync_copy(buf_v, sh_idx.at[pl.ds(pl.multiple_of(s * 64, 8), 64)])
        plsc.subcore_barrier()

        @pl.when(sub == 0)
        def _():
            pltpu.sync_copy(
                sh_idx.at[pl.ds(pl.multiple_of(dir_ * 512, 8), 512)], mb_v)
            pltpu.sync_copy(
                sh_cnt.at[pl.ds(pl.multiple_of(dir_ * 128, 8), 128)], mc_v)
            for q in range(4):
                gbuf_v[pl.ds(q * 16, 16)] = jnp.zeros((16,), jnp.int32)
            running = jnp.int32(0)
            for t in range(8):
                cs = mc_v[pl.ds(t * 16, 16)][0]
                for q in range(4):
                    vals = mb_v[pl.ds(t * 64 + q * 16, 16)]
                    valid = (lax.iota(jnp.int32, 16) + q * 16) < cs
                    vi = jnp.where(valid, jnp.int32(1), jnp.int32(0))
                    csum = plsc.cumsum(vi)
                    pos = running + csum - 1
                    m2 = valid & (pos < NINST)
                    plsc.store_scatter(gbuf_v, [pos], vals, mask=m2)
                    running = running + csum[15]
            pltpu.async_copy(x_hbm.at[gbuf_v], rows_v, sem).wait()
            pltpu.sync_copy(
                rows_v,
                out_hbm.at[pl.ds(pl.multiple_of(r_thr * 64, 8), 64)])

    return sc_sel


def _loss_body(xg_ref, wfct_ref, bfc_ref, wi_ref, binst_ref, label_ref,
               loss_ref):
    h = jnp.maximum(
        lax.dot_general(xg_ref[...], wfct_ref[...], (((1,), (1,)), ((), ())),
                        preferred_element_type=jnp.float32)
        + bfc_ref[...], 0.0)
    lab = label_ref[0, 0]
    total = jnp.zeros((1, 1), jnp.float32)

    def branch_sum(hp, c, tgt_is1):
        lg = lax.dot_general(hp, wi_ref[2 * c:2 * c + 2, :],
                             (((1,), (1,)), ((), ())),
                             preferred_element_type=jnp.float32)
        lg = lg + binst_ref[c:c + 1, :]
        du = lg[:, 0:1] + (1.0 if tgt_is1 else 0.0)
        dv = lg[:, 1:2] + (0.0 if tgt_is1 else 1.0)
        mx = jnp.maximum(du, dv)
        lse = mx + jnp.log(jnp.exp(du - mx) + jnp.exp(dv - mx))
        s_y = lg[:, 1:2] if tgt_is1 else lg[:, 0:1]
        return jnp.sum(lse - s_y, axis=0, keepdims=True)

    for c in range(NCLS):
        top = h[64 * c:64 * c + NINST, :]
        bot = h[128 + 64 * c:128 + 64 * c + NINST, :]
        loss_c = (branch_sum(top, c, True)
                  + branch_sum(bot, c, False)) / (2 * NINST)
        total = total + jnp.where(lab == c, loss_c,
                                  jnp.zeros((1, 1), jnp.float32))
    loss_ref[...] = total


def _loss(xg, w_fct, b_fc2, wi, binst, label2):
    return pl.pallas_call(
        _loss_body,
        out_shape=jax.ShapeDtypeStruct((1, 1), jnp.float32),
    )(xg, w_fct, b_fc2, wi, binst, label2)


def kernel(x, label, W_fc, b_fc, Wa, ba, Wb, bb, Wc, bc, Wbag, bbag, Winst,
           binst):
    w_fct = W_fc
    wat = Wa
    wbt = Wb
    b_fc2 = b_fc.reshape(1, HID)
    ba2 = ba.reshape(1, PROJ)
    bb2 = bb.reshape(1, PROJ)
    bc2 = bc.reshape(NCLS, 1)
    bbag2 = bbag.reshape(NCLS, 1)
    wi = Winst.reshape(2 * NCLS, HID)
    label2 = label.reshape(1, 1)

    a_pad, logits, y_prob, yhat2 = _stage1(
        x, w_fct, b_fc2, wat, ba2, wbt, bb2, Wc, bc2, Wbag, bbag2)
    a2n = a_pad[:, :N]

    thr = _radix(a2n)
    xg = _make_sc_select_gather()(a2n.reshape(NCLS * N), thr.reshape(-1), x)

    loss2 = _loss(xg, w_fct, b_fc2, wi, binst, label2)

    return (logits.reshape(1, NCLS), y_prob.reshape(1, NCLS),
            yhat2.reshape((1,)), a2n, loss2.reshape(()))


# speedup vs baseline: 1.0075x; 1.0075x over previous
import functools

import jax
import jax.numpy as jnp
from jax import lax
from jax.experimental import pallas as pl
from jax.experimental.pallas import tpu as pltpu
from jax.experimental.pallas import tpu_sc as plsc

N = 50000
ENC = 1024
HID = 512
PROJ = 256
NCLS = 2
NINST = 50

TILE = 4096
NSTEPS = (N + TILE - 1) // TILE
APAD = NSTEPS * TILE

GATHER_B = 256


def _stage1_body(x_ref, wfct_ref, bfc_ref, wat_ref, ba_ref, wbt_ref, bb_ref,
                 wc_ref, bc_ref, wbag_ref, bbag_ref,
                 a_out_ref, logits_ref, prob_ref, yhat_ref,
                 m_ref, z_ref, macc_ref):
    i = pl.program_id(0)

    @pl.when(i == 0)
    def _():
        m_ref[...] = jnp.full((NCLS, 1), -jnp.inf, jnp.float32)
        z_ref[...] = jnp.zeros((NCLS, 1), jnp.float32)
        macc_ref[...] = jnp.zeros((NCLS, HID), jnp.float32)

    ct = (((1,), (1,)), ((), ()))
    h = jnp.maximum(
        lax.dot_general(x_ref[...], wfct_ref[...], ct,
                        preferred_element_type=jnp.float32)
        + bfc_ref[...], 0.0)
    a = jnp.tanh(
        lax.dot_general(h, wat_ref[...], ct,
                        preferred_element_type=jnp.float32)
        + ba_ref[...])
    g = jax.nn.sigmoid(
        lax.dot_general(h, wbt_ref[...], ct,
                        preferred_element_type=jnp.float32)
        + bb_ref[...])
    ag = a * g
    a_t = lax.dot_general(wc_ref[...], ag, (((1,), (1,)), ((), ())),
                          preferred_element_type=jnp.float32) + bc_ref[...]
    a_out_ref[...] = a_t

    base = i * TILE
    cmask = (lax.broadcasted_iota(jnp.int32, (NCLS, TILE), 1) + base) < N
    rmask = (lax.broadcasted_iota(jnp.int32, (TILE, 1), 0) + base) < N
    am = jnp.where(cmask, a_t, -jnp.inf)
    hm = jnp.where(rmask, h, 0.0)

    t_max = jnp.max(am, axis=1, keepdims=True)
    m_old = m_ref[...]
    m_new = jnp.maximum(m_old, t_max)
    alpha = jnp.exp(m_old - m_new)
    p = jnp.exp(am - m_new)
    z_ref[...] = z_ref[...] * alpha + jnp.sum(p, axis=1, keepdims=True)
    contrib = lax.dot_general(p, hm, (((1,), (0,)), ((), ())),
                              preferred_element_type=jnp.float32)
    macc_ref[...] = macc_ref[...] * alpha + contrib
    m_ref[...] = m_new

    @pl.when(i == NSTEPS - 1)
    def _():
        mn = macc_ref[...] / z_ref[...]
        bag = jnp.sum(mn * wbag_ref[...], axis=1, keepdims=True) + bbag_ref[...]
        logits_ref[...] = bag
        mx = jnp.max(bag, axis=0, keepdims=True)
        e = jnp.exp(bag - mx)
        prob_ref[...] = e / jnp.sum(e, axis=0, keepdims=True)
        yhat_ref[...] = jnp.where(bag[1:2, 0:1] > bag[0:1, 0:1], 1, 0
                                  ).astype(jnp.int32)


def _stage1(x, w_fct, b_fc2, wat, ba2, wbt, bb2, wc, bc2, wbag, bbag2):
    full = lambda shape: pl.BlockSpec(shape, lambda i: (0, 0))
    return pl.pallas_call(
        _stage1_body,
        grid=(NSTEPS,),
        in_specs=[
            pl.BlockSpec((TILE, ENC), lambda i: (i, 0)),
            full((HID, ENC)), full((1, HID)),
            full((PROJ, HID)), full((1, PROJ)),
            full((PROJ, HID)), full((1, PROJ)),
            full((NCLS, PROJ)), full((NCLS, 1)),
            full((NCLS, HID)), full((NCLS, 1)),
        ],
        out_specs=[
            pl.BlockSpec((NCLS, TILE), lambda i: (0, i)),
            full((NCLS, 1)), full((NCLS, 1)), full((1, 1)),
        ],
        out_shape=[
            jax.ShapeDtypeStruct((NCLS, APAD), jnp.float32),
            jax.ShapeDtypeStruct((NCLS, 1), jnp.float32),
            jax.ShapeDtypeStruct((NCLS, 1), jnp.float32),
            jax.ShapeDtypeStruct((1, 1), jnp.int32),
        ],
        scratch_shapes=[
            pltpu.VMEM((NCLS, 1), jnp.float32),
            pltpu.VMEM((NCLS, 1), jnp.float32),
            pltpu.VMEM((NCLS, HID), jnp.float32),
        ],
    )(x, w_fct, b_fc2, wat, ba2, wbt, bb2, wc, bc2, wbag, bbag2)


def _radix_body(a_ref, thr_ref):
    HI = jnp.uint32(0x80000000)
    av = a_ref[...]
    b = lax.bitcast_convert_type(av, jnp.uint32)
    keyt = jnp.where(b >= HI, ~b, b ^ HI)
    keys = jnp.concatenate([keyt, ~keyt], axis=0)

    def body(t, p):
        bit = (31 - t).astype(jnp.uint32)
        cand = p | lax.shift_left(jnp.uint32(1), bit)
        cnt = jnp.sum((keys >= cand).astype(jnp.int32), axis=1, keepdims=True)
        return jnp.where(cnt >= NINST, cand, p)

    p = lax.fori_loop(0, 32, body, jnp.zeros((2 * NCLS, 1), jnp.uint32))
    rowi = lax.broadcasted_iota(jnp.int32, (2 * NCLS, 1), 0)
    ktop = jnp.where(rowi >= NCLS, ~p, p)
    bits = jnp.where(ktop >= HI, ktop ^ HI, ~ktop)
    tf = lax.bitcast_convert_type(bits, jnp.float32)
    thr_ref[...] = jnp.broadcast_to(tf, (2 * NCLS, 16))


def _radix(a2n):
    return pl.pallas_call(
        _radix_body,
        out_shape=jax.ShapeDtypeStruct((2 * NCLS, 16), jnp.float32),
    )(a2n)


SC_CH = 6272
SC_CH_LAST = N - 7 * SC_CH
SC_NV = SC_CH // 16


def _make_sc_select_gather():
    mesh = plsc.VectorSubcoreMesh(core_axis_name="c", subcore_axis_name="s")

    @functools.partial(
        pl.kernel, mesh=mesh,
        out_type=jax.ShapeDtypeStruct((GATHER_B, ENC), jnp.float32),
        compiler_params=pltpu.CompilerParams(needs_layout_passes=False),
        scratch_types=[
            pltpu.VMEM((SC_CH,), jnp.float32),
            pltpu.VMEM((16,), jnp.float32),
            pltpu.VMEM((64,), jnp.int32),
            pltpu.VMEM((16,), jnp.int32),
            pltpu.VMEM((512,), jnp.int32),
            pltpu.VMEM((128,), jnp.int32),
            pltpu.VMEM((64,), jnp.int32),
            pltpu.VMEM((64, ENC), jnp.float32),
            pltpu.VMEM_SHARED((1024,), jnp.int32),
            pltpu.VMEM_SHARED((256,), jnp.int32),
            pltpu.SemaphoreType.DMA,
        ],
    )
    def sc_sel(a_hbm, thr_hbm, x_hbm, out_hbm, a_v, thr_v, buf_v, cnt_v,
               mb_v, mc_v, gbuf_v, rows_v, sh_idx, sh_cnt, sem):
        c = lax.axis_index("c")
        s = lax.axis_index("s")
        dir_ = s // 8
        sub = s % 8
        base = sub * SC_CH
        r_thr = dir_ * NCLS + c
        a_off = pl.multiple_of(c * N + base, 8)

        @pl.when(sub < 7)
        def _():
            pltpu.sync_copy(a_hbm.at[pl.ds(a_off, SC_CH)], a_v)

        @pl.when(sub == 7)
        def _():
            pltpu.sync_copy(a_hbm.at[pl.ds(a_off, SC_CH_LAST)],
                            a_v.at[pl.ds(0, SC_CH_LAST)])

        pltpu.sync_copy(thr_hbm.at[pl.ds(pl.multiple_of(r_thr * 16, 8), 16)],
                        thr_v)
        is_bot = dir_ == 1

        def body(j, cnt):
            v = a_v[pl.ds(j * 16, 16)]
            tf = thr_v[...]
            gidx = base + j * 16 + lax.iota(jnp.int32, 16)
            selm = jnp.where(is_bot, v <= tf, v >= tf)
            m = selm & (gidx < N)
            vi = jnp.where(m, jnp.int32(1), jnp.int32(0))
            csum = plsc.cumsum(vi)
            pos = cnt + csum - 1
            m2 = m & (pos < 64)
            plsc.store_scatter(buf_v, [pos], gidx, mask=m2)
            return cnt + csum[15]

        cntf = lax.fori_loop(0, SC_NV, body, jnp.int32(0), unroll=14)

        cnt_v[...] = jnp.full((16,), cntf, jnp.int32)
        pltpu.sync_copy(cnt_v, sh_cnt.at[pl.ds(pl.multiple_of(s * 16, 8), 16)])
        pltpu.sync_copy(buf_v, sh_idx.at[pl.ds(pl.multiple_of(s * 64, 8), 64)])
        plsc.subcore_barrier()

        @pl.when(sub == 0)
        def _():
            pltpu.sync_copy(
                sh_idx.at[pl.ds(pl.multiple_of(dir_ * 512, 8), 512)], mb_v)
            pltpu.sync_copy(
                sh_cnt.at[pl.ds(pl.multiple_of(dir_ * 128, 8), 128)], mc_v)
            for q in range(4):
                gbuf_v[pl.ds(q * 16, 16)] = jnp.zeros((16,), jnp.int32)
            running = jnp.int32(0)
            for t in range(8):
                cs = mc_v[pl.ds(t * 16, 16)][0]
                for q in range(4):
                    vals = mb_v[pl.ds(t * 64 + q * 16, 16)]
                    valid = (lax.iota(jnp.int32, 16) + q * 16) < cs
                    vi = jnp.where(valid, jnp.int32(1), jnp.int32(0))
                    csum = plsc.cumsum(vi)
                    pos = running + csum - 1
                    m2 = valid & (pos < NINST)
                    plsc.store_scatter(gbuf_v, [pos], vals, mask=m2)
                    running = running + csum[15]
            pltpu.async_copy(x_hbm.at[gbuf_v], rows_v, sem).wait()
            pltpu.sync_copy(
                rows_v,
                out_hbm.at[pl.ds(pl.multiple_of(r_thr * 64, 8), 64)])

    return sc_sel


def _loss_body(xg_ref, wfct_ref, bfc_ref, wi_ref, binst_ref, label_ref,
               loss_ref):
    h = jnp.maximum(
        lax.dot_general(xg_ref[...], wfct_ref[...], (((1,), (1,)), ((), ())),
                        preferred_element_type=jnp.float32)
        + bfc_ref[...], 0.0)
    lab = label_ref[0, 0]
    total = jnp.zeros((1, 1), jnp.float32)

    def branch_sum(hp, c, tgt_is1):
        lg = lax.dot_general(hp, wi_ref[2 * c:2 * c + 2, :],
                             (((1,), (1,)), ((), ())),
                             preferred_element_type=jnp.float32)
        lg = lg + binst_ref[c:c + 1, :]
        du = lg[:, 0:1] + (1.0 if tgt_is1 else 0.0)
        dv = lg[:, 1:2] + (0.0 if tgt_is1 else 1.0)
        mx = jnp.maximum(du, dv)
        lse = mx + jnp.log(jnp.exp(du - mx) + jnp.exp(dv - mx))
        s_y = lg[:, 1:2] if tgt_is1 else lg[:, 0:1]
        return jnp.sum(lse - s_y, axis=0, keepdims=True)

    for c in range(NCLS):
        top = h[64 * c:64 * c + NINST, :]
        bot = h[128 + 64 * c:128 + 64 * c + NINST, :]
        loss_c = (branch_sum(top, c, True)
                  + branch_sum(bot, c, False)) / (2 * NINST)
        total = total + jnp.where(lab == c, loss_c,
                                  jnp.zeros((1, 1), jnp.float32))
    loss_ref[...] = total


def _loss(xg, w_fct, b_fc2, wi, binst, label2):
    return pl.pallas_call(
        _loss_body,
        out_shape=jax.ShapeDtypeStruct((1, 1), jnp.float32),
    )(xg, w_fct, b_fc2, wi, binst, label2)


def kernel(x, label, W_fc, b_fc, Wa, ba, Wb, bb, Wc, bc, Wbag, bbag, Winst,
           binst):
    w_fct = W_fc
    wat = Wa
    wbt = Wb
    b_fc2 = b_fc.reshape(1, HID)
    ba2 = ba.reshape(1, PROJ)
    bb2 = bb.reshape(1, PROJ)
    bc2 = bc.reshape(NCLS, 1)
    bbag2 = bbag.reshape(NCLS, 1)
    wi = Winst.reshape(2 * NCLS, HID)
    label2 = label.reshape(1, 1)

    a_pad, logits, y_prob, yhat2 = _stage1(
        x, w_fct, b_fc2, wat, ba2, wbt, bb2, Wc, bc2, Wbag, bbag2)
    a2n = a_pad[:, :N]

    thr = _radix(a2n)
    xg = _make_sc_select_gather()(a2n.reshape(NCLS * N), thr.reshape(-1), x)

    loss2 = _loss(xg, w_fct, b_fc2, wi, binst, label2)

    return (logits.reshape(1, NCLS), y_prob.reshape(1, NCLS),
            yhat2.reshape((1,)), a2n, loss2.reshape(()))
